# Initial kernel scaffold; baseline (speedup 1.0000x reference)
import functools
import jax
import jax.numpy as jnp
from jax import lax
from jax.experimental import pallas as pl
from jax.experimental.pallas import tpu as pltpu
from jax.experimental.pallas import tpu_sc as plsc

N, E, D, H = 10000, 320000, 128, 128
NC, NS, L = 2, 16, 16
NT = NC * NS
CHUNK = 128
CPT = 80
E_PAD = NT * CPT * CHUNK
CPT_DEG = E_PAD // (NS * CHUNK)
ROWS_PER_TILE = N // NS

_mesh = plsc.VectorSubcoreMesh(core_axis_name="c", subcore_axis_name="s")


def _mlp_body(ee_ref, W11_ref, b11_ref, W21_ref, b21_ref,
              W12_ref, b12_ref, W22_ref, b22_ref, w1_ref, w2_ref):
    ee = ee_ref[...]
    dn = (((1,), (1,)), ((), ()))
    h1 = jnp.maximum(lax.dot_general(ee, W11_ref[...], dn,
                                     preferred_element_type=jnp.float32)
                     + b11_ref[...], 0.0)
    w1_ref[...] = jax.nn.sigmoid(jnp.sum(h1 * W21_ref[...], axis=1)
                                 + b21_ref[0, 0])
    h2 = jnp.maximum(lax.dot_general(ee, W12_ref[...], dn,
                                     preferred_element_type=jnp.float32)
                     + b12_ref[...], 0.0)
    w2_ref[...] = jax.nn.sigmoid(jnp.sum(h2 * W22_ref[...], axis=1)
                                 + b22_ref[0, 0])


def _edge_mlp(edge_embed, W11, b11, W21, b21, W12, b12, W22, b22):
    BE = 6400
    grid = E // BE
    full = pl.BlockSpec((H, H), lambda i: (0, 0))
    vec = pl.BlockSpec((1, H), lambda i: (0, 0))
    return pl.pallas_call(
        _mlp_body,
        grid=(grid,),
        in_specs=[pl.BlockSpec((BE, H), lambda i: (i, 0)),
                  full, vec, vec, pl.BlockSpec((1, 1), lambda i: (0, 0)),
                  full, vec, vec, pl.BlockSpec((1, 1), lambda i: (0, 0))],
        out_specs=[pl.BlockSpec((BE,), lambda i: (i,)),
                   pl.BlockSpec((BE,), lambda i: (i,))],
        out_shape=[jax.ShapeDtypeStruct((E,), jnp.float32),
                   jax.ShapeDtypeStruct((E,), jnp.float32)],
    )(edge_embed, W11, b11.reshape(1, H), W21, b21.reshape(1, 1),
      W12, b12.reshape(1, H), W22, b22.reshape(1, 1))


def _matmul_body(x_ref, W_ref, o_ref):
    o_ref[...] = lax.dot_general(x_ref[...], W_ref[...],
                                 (((1,), (1,)), ((), ())),
                                 preferred_element_type=jnp.float32)


def _node_matmul(x, Wl):
    return pl.pallas_call(
        _matmul_body,
        out_shape=jax.ShapeDtypeStruct((N, D), jnp.float32),
    )(x, Wl)


def _dinv_body(deg_ref, o_ref):
    o_ref[...] = lax.rsqrt(deg_ref[...] + 1.0)


def _dinv(deg):
    return pl.pallas_call(
        _dinv_body,
        out_shape=jax.ShapeDtypeStruct((NC, N), jnp.float32),
    )(deg)


def _fin1_body(p_ref, xl_ref, dinv_ref, bias_ref, Wl2_ref, o_ref):
    dsq = dinv_ref[...] * dinv_ref[...]
    h = p_ref[0:N, :] + p_ref[N:2 * N, :] + dsq * xl_ref[...] + bias_ref[...]
    h = jnp.maximum(h, 0.0)
    o_ref[...] = lax.dot_general(h, Wl2_ref[...], (((1,), (1,)), ((), ())),
                                 preferred_element_type=jnp.float32)


def _fin1(p, xl, dinv_col, bias, Wl2):
    return pl.pallas_call(
        _fin1_body,
        out_shape=jax.ShapeDtypeStruct((N, D), jnp.float32),
    )(p, xl, dinv_col, bias.reshape(1, D), Wl2)


def _fin2_body(p_ref, xl_ref, dinv_ref, bias_ref, o_ref):
    dsq = dinv_ref[...] * dinv_ref[...]
    h = p_ref[0:N, :] + p_ref[N:2 * N, :] + dsq * xl_ref[...] + bias_ref[...]
    o_ref[...] = jnp.maximum(h + h, 0.0)


def _fin2(p, xl, dinv_col, bias):
    return pl.pallas_call(
        _fin2_body,
        out_shape=jax.ShapeDtypeStruct((N, D), jnp.float32),
    )(p, xl, dinv_col, bias.reshape(1, D))


def _deg_body(w_hbm, cols_hbm, out_hbm, cols_v, w_v, zbuf, acc, sem):
    c = lax.axis_index("c")
    s = lax.axis_index("s")
    pltpu.sync_copy(cols_hbm.at[s], cols_v)
    pltpu.sync_copy(w_hbm.at[c, s], w_v)

    def _zb(i, carry):
        zbuf[pl.ds(i * L, L)] = jnp.zeros((L,), jnp.float32)
        return carry
    lax.fori_loop(0, 63, _zb, 0)

    @pl.when(s == 0)
    def _():
        def _zc(i, carry):
            pltpu.sync_copy(zbuf.at[pl.ds(0, 1000)],
                            acc.at[pl.ds(i * 1000, 1000)])
            return carry
        lax.fori_loop(0, 10, _zc, 0)

    plsc.subcore_barrier()

    def _sc(j, carry):
        pltpu.sync_copy(w_v.at[j], acc.at[cols_v.at[j]], add=True)
        return carry
    lax.fori_loop(0, CPT_DEG, _sc, 0)

    plsc.subcore_barrier()

    @pl.when(s == 0)
    def _():
        pltpu.sync_copy(acc, out_hbm.at[c])


def _deg(w_deg, cols_deg):
    return pl.kernel(
        _deg_body,
        out_type=jax.ShapeDtypeStruct((NC, N), jnp.float32),
        mesh=_mesh,
        scratch_types=[
            pltpu.VMEM((CPT_DEG, CHUNK), jnp.int32),
            pltpu.VMEM((CPT_DEG, CHUNK), jnp.float32),
            pltpu.VMEM((1008,), jnp.float32),
            pltpu.VMEM_SHARED((N,), jnp.float32),
            pltpu.SemaphoreType.DMA,
        ],
    )(w_deg, cols_deg)


def _seg_body(xl_hbm, rows_hbm, cols_hbm, w_hbm, dinv_hbm, out_hbm,
              rows_v, cols_v, norm_v, dinv_v, gbuf, acc, gsem):
    c = lax.axis_index("c")
    s = lax.axis_index("s")
    wid = c * NS + s
    pltpu.sync_copy(rows_hbm.at[wid], rows_v)
    pltpu.sync_copy(cols_hbm.at[wid], cols_v)
    pltpu.sync_copy(w_hbm.at[wid], norm_v)
    pltpu.sync_copy(dinv_hbm, dinv_v)

    def _zb(i, carry):
        for k in range(D // L):
            gbuf[i, pl.ds(k * L, L)] = jnp.zeros((L,), jnp.float32)
        return carry
    lax.fori_loop(0, CHUNK, _zb, 0)
    for r in range(5):
        pltpu.sync_copy(gbuf.at[pl.ds(0, 125)],
                        acc.at[pl.ds(s * ROWS_PER_TILE + r * 125, 125)])

    def _nb(j, carry):
        for k in range(CHUNK // L):
            sl = pl.ds(k * L, L)
            r16 = rows_v[j, sl]
            c16 = cols_v[j, sl]
            dr = plsc.load_gather(dinv_v, [r16])
            dc = plsc.load_gather(dinv_v, [c16])
            norm_v[j, sl] = norm_v[j, sl] * dr * dc
        return carry
    lax.fori_loop(0, CPT, _nb, 0)

    plsc.subcore_barrier()

    def _mb(j, carry):
        pltpu.async_copy(xl_hbm.at[rows_v.at[j]], gbuf, gsem).wait()

        def _eb(e, carry2):
            nv = norm_v[j, e]
            for k in range(D // L):
                sl = pl.ds(k * L, L)
                gbuf[e, sl] = gbuf[e, sl] * nv
            return carry2
        lax.fori_loop(0, CHUNK, _eb, 0)
        pltpu.sync_copy(gbuf, acc.at[cols_v.at[j]], add=True)
        return carry
    lax.fori_loop(0, CPT, _mb, 0)

    plsc.subcore_barrier()

    pltpu.sync_copy(acc.at[pl.ds(s * ROWS_PER_TILE, ROWS_PER_TILE)],
                    out_hbm.at[pl.ds(c * N + s * ROWS_PER_TILE,
                                     ROWS_PER_TILE)])


def _seg(xl, rows3, cols3, w3, dinv):
    return pl.kernel(
        _seg_body,
        out_type=jax.ShapeDtypeStruct((NC * N, D), jnp.float32),
        mesh=_mesh,
        scratch_types=[
            pltpu.VMEM((CPT, CHUNK), jnp.int32),
            pltpu.VMEM((CPT, CHUNK), jnp.int32),
            pltpu.VMEM((CPT, CHUNK), jnp.float32),
            pltpu.VMEM((N,), jnp.float32),
            pltpu.VMEM((CHUNK, D), jnp.float32),
            pltpu.VMEM_SHARED((N, D), jnp.float32),
            pltpu.SemaphoreType.DMA,
        ],
    )(xl, rows3, cols3, w3, dinv)


def kernel(x, edge_index, edge_embed, Wl1, W11, b11, W21, b21, bias1,
           Wl2, W12, b12, W22, b22, bias2):
    row = edge_index[0]
    col = edge_index[1]
    pad = E_PAD - E
    ipad = jnp.zeros((pad,), jnp.int32)
    rows3 = jnp.concatenate([row, ipad]).reshape(NT, CPT, CHUNK)
    colp = jnp.concatenate([col, ipad])
    cols3 = colp.reshape(NT, CPT, CHUNK)
    cols_deg = colp.reshape(NS, CPT_DEG, CHUNK)

    w1, w2 = _edge_mlp(edge_embed, W11, b11, W21, b21, W12, b12, W22, b22)
    fpad = jnp.zeros((pad,), jnp.float32)
    w1p = jnp.concatenate([w1, fpad])
    w2p = jnp.concatenate([w2, fpad])
    w_deg = jnp.stack([w1p, w2p]).reshape(NC, NS, CPT_DEG, CHUNK)

    deg = _deg(w_deg, cols_deg)
    dinv = _dinv(deg)
    dinv1 = dinv[0]
    dinv2 = dinv[1]

    xl1 = _node_matmul(x, Wl1)
    p1 = _seg(xl1, rows3, cols3, w1p.reshape(NT, CPT, CHUNK), dinv1)
    xl2 = _fin1(p1, xl1, dinv1.reshape(N, 1), bias1, Wl2)
    p2 = _seg(xl2, rows3, cols3, w2p.reshape(NT, CPT, CHUNK), dinv2)
    return _fin2(p2, xl2, dinv2.reshape(N, 1), bias2)


# speedup vs baseline: 8.8545x; 8.8545x over previous
import functools
import jax
import jax.numpy as jnp
from jax import lax
from jax.experimental import pallas as pl
from jax.experimental.pallas import tpu as pltpu
from jax.experimental.pallas import tpu_sc as plsc

N, E, D, H = 10000, 320000, 128, 128
NC, NS, L = 2, 16, 16
NT = NC * NS
CHUNK = 128
CPT = 80
E_PAD = NT * CPT * CHUNK
CPT_DEG = E_PAD // (NS * CHUNK)
ROWS_PER_TILE = N // NS

_mesh = plsc.VectorSubcoreMesh(core_axis_name="c", subcore_axis_name="s")


def _mlp_body(ee_ref, W11_ref, b11_ref, W21_ref, b21_ref,
              W12_ref, b12_ref, W22_ref, b22_ref, w1_ref, w2_ref):
    ee = ee_ref[...]
    dn = (((1,), (1,)), ((), ()))
    h1 = jnp.maximum(lax.dot_general(ee, W11_ref[...], dn,
                                     preferred_element_type=jnp.float32)
                     + b11_ref[...], 0.0)
    w1_ref[...] = jax.nn.sigmoid(jnp.sum(h1 * W21_ref[...], axis=1)
                                 + b21_ref[0, 0])[:, None]
    h2 = jnp.maximum(lax.dot_general(ee, W12_ref[...], dn,
                                     preferred_element_type=jnp.float32)
                     + b12_ref[...], 0.0)
    w2_ref[...] = jax.nn.sigmoid(jnp.sum(h2 * W22_ref[...], axis=1)
                                 + b22_ref[0, 0])[:, None]


def _edge_mlp(edge_embed, W11, b11, W21, b21, W12, b12, W22, b22):
    BE = 6400
    grid = E // BE
    full = pl.BlockSpec((H, H), lambda i: (0, 0))
    vec = pl.BlockSpec((1, H), lambda i: (0, 0))
    return pl.pallas_call(
        _mlp_body,
        grid=(grid,),
        in_specs=[pl.BlockSpec((BE, H), lambda i: (i, 0)),
                  full, vec, vec, pl.BlockSpec((1, 1), lambda i: (0, 0)),
                  full, vec, vec, pl.BlockSpec((1, 1), lambda i: (0, 0))],
        out_specs=[pl.BlockSpec((BE, 1), lambda i: (i, 0)),
                   pl.BlockSpec((BE, 1), lambda i: (i, 0))],
        out_shape=[jax.ShapeDtypeStruct((E, 1), jnp.float32),
                   jax.ShapeDtypeStruct((E, 1), jnp.float32)],
    )(edge_embed, W11, b11.reshape(1, H), W21, b21.reshape(1, 1),
      W12, b12.reshape(1, H), W22, b22.reshape(1, 1))


def _matmul_body(x_ref, W_ref, o_ref):
    o_ref[...] = lax.dot_general(x_ref[...], W_ref[...],
                                 (((1,), (1,)), ((), ())),
                                 preferred_element_type=jnp.float32)


def _node_matmul(x, Wl):
    return pl.pallas_call(
        _matmul_body,
        out_shape=jax.ShapeDtypeStruct((N, D), jnp.float32),
    )(x, Wl)


def _dinv_body(deg_ref, o_ref):
    o_ref[...] = lax.rsqrt(deg_ref[...] + 1.0)


def _dinv(deg):
    return pl.pallas_call(
        _dinv_body,
        out_shape=jax.ShapeDtypeStruct((NC * N,), jnp.float32),
    )(deg)


def _fin1_body(p_ref, xl_ref, dinv_ref, bias_ref, Wl2_ref, o_ref):
    dsq = dinv_ref[...] * dinv_ref[...]
    h = p_ref[0:N, :] + p_ref[N:2 * N, :] + dsq * xl_ref[...] + bias_ref[...]
    h = jnp.maximum(h, 0.0)
    o_ref[...] = lax.dot_general(h, Wl2_ref[...], (((1,), (1,)), ((), ())),
                                 preferred_element_type=jnp.float32)


def _fin1(p, xl, dinv_col, bias, Wl2):
    return pl.pallas_call(
        _fin1_body,
        out_shape=jax.ShapeDtypeStruct((N, D), jnp.float32),
    )(p, xl, dinv_col, bias.reshape(1, D), Wl2)


def _fin2_body(p_ref, xl_ref, dinv_ref, bias_ref, o_ref):
    dsq = dinv_ref[...] * dinv_ref[...]
    h = p_ref[0:N, :] + p_ref[N:2 * N, :] + dsq * xl_ref[...] + bias_ref[...]
    o_ref[...] = jnp.maximum(h + h, 0.0)


def _fin2(p, xl, dinv_col, bias):
    return pl.pallas_call(
        _fin2_body,
        out_shape=jax.ShapeDtypeStruct((N, D), jnp.float32),
    )(p, xl, dinv_col, bias.reshape(1, D))


def _deg_body(w_hbm, cols_hbm, out_hbm, cols_v, w_v, zbuf, vbuf, acc, sem):
    c = lax.axis_index("c")
    s = lax.axis_index("s")
    pltpu.sync_copy(cols_hbm.at[s], cols_v)
    pltpu.sync_copy(w_hbm.at[c, s], w_v)

    def _zb(i, carry):
        zbuf[pl.ds(i * L, L)] = jnp.zeros((L,), jnp.float32)
        return carry
    lax.fori_loop(0, 63, _zb, 0)

    @pl.when(s == 0)
    def _():
        def _zc(i, carry):
            pltpu.sync_copy(zbuf.at[pl.ds(0, 1000)],
                            acc.at[pl.ds(i * 1000, 1000)])
            return carry
        lax.fori_loop(0, 10, _zc, 0)

    plsc.subcore_barrier()

    def _sc(j, carry):
        pltpu.sync_copy(w_v.at[j], acc.at[cols_v.at[j]], add=True)
        return carry
    lax.fori_loop(0, CPT_DEG, _sc, 0)

    plsc.subcore_barrier()

    @pl.when(s == 0)
    def _():
        pltpu.sync_copy(acc, vbuf)
        pltpu.sync_copy(vbuf, out_hbm.at[pl.ds(c * N, N)])


def _deg(w_deg, cols_deg):
    return pl.kernel(
        _deg_body,
        out_type=jax.ShapeDtypeStruct((NC * N,), jnp.float32),
        mesh=_mesh,
        scratch_types=[
            pltpu.VMEM((CPT_DEG, CHUNK), jnp.int32),
            pltpu.VMEM((CPT_DEG, CHUNK), jnp.float32),
            pltpu.VMEM((1008,), jnp.float32),
            pltpu.VMEM((N,), jnp.float32),
            pltpu.VMEM_SHARED((N,), jnp.float32),
            pltpu.SemaphoreType.DMA,
        ],
    )(w_deg, cols_deg)


def _seg_body(xl_hbm, rows_hbm, cols_hbm, w_hbm, dinv_hbm, out_hbm,
              rows_v, cols_v, w_v, drbuf, dcbuf, nbuf, gbuf, acc,
              gsem, nsem):
    c = lax.axis_index("c")
    s = lax.axis_index("s")
    wid = c * NS + s
    pltpu.sync_copy(rows_hbm.at[wid], rows_v)
    pltpu.sync_copy(cols_hbm.at[wid], cols_v)
    pltpu.sync_copy(w_hbm.at[wid], w_v)

    def _zb(i, carry):
        for k in range(D // L):
            gbuf[i, pl.ds(k * L, L)] = jnp.zeros((L,), jnp.float32)
        return carry
    lax.fori_loop(0, CHUNK, _zb, 0)
    base = s * 624

    @pl.when(s < NS - 1)
    def _():
        for m, ln in ((0, 128), (128, 128), (256, 128), (384, 128),
                      (512, 112)):
            pltpu.sync_copy(gbuf.at[pl.ds(0, ln)],
                            acc.at[pl.ds(base + m, ln)])

    @pl.when(s == NS - 1)
    def _():
        for m in range(5):
            pltpu.sync_copy(gbuf.at[pl.ds(0, 128)],
                            acc.at[pl.ds(base + m * 128, 128)])

    plsc.subcore_barrier()

    def _mb(j, carry):
        gd = pltpu.async_copy(xl_hbm.at[rows_v.at[j]], gbuf, gsem)
        rd = pltpu.async_copy(dinv_hbm.at[rows_v.at[j]], drbuf, nsem)
        cd = pltpu.async_copy(dinv_hbm.at[cols_v.at[j]], dcbuf, nsem)
        rd.wait()
        cd.wait()
        for k in range(CHUNK // L):
            sl = pl.ds(k * L, L)
            nbuf[sl] = w_v[j, sl] * drbuf[sl] * dcbuf[sl]
        gd.wait()

        def _eb(g, carry2):
            nv16 = nbuf[pl.ds(g * L, L)]
            for l in range(L):
                e = g * L + l
                nv = nv16[l]
                for k in range(D // L):
                    sl = pl.ds(k * L, L)
                    gbuf[e, sl] = gbuf[e, sl] * nv
            return carry2
        lax.fori_loop(0, CHUNK // L, _eb, 0)
        pltpu.sync_copy(gbuf, acc.at[cols_v.at[j]], add=True)
        return carry
    lax.fori_loop(0, CPT, _mb, 0)

    plsc.subcore_barrier()

    @pl.when(s < NS - 1)
    def _():
        pltpu.sync_copy(acc.at[pl.ds(base, 624)],
                        out_hbm.at[pl.ds(c * N + base, 624)])

    @pl.when(s == NS - 1)
    def _():
        pltpu.sync_copy(acc.at[pl.ds(base, 640)],
                        out_hbm.at[pl.ds(c * N + base, 640)])


def _seg(xl, rows3, cols3, w3, dinv):
    return pl.kernel(
        _seg_body,
        out_type=jax.ShapeDtypeStruct((NC * N, D), jnp.float32),
        mesh=_mesh,
        scratch_types=[
            pltpu.VMEM((CPT, CHUNK), jnp.int32),
            pltpu.VMEM((CPT, CHUNK), jnp.int32),
            pltpu.VMEM((CPT, CHUNK), jnp.float32),
            pltpu.VMEM((CHUNK,), jnp.float32),
            pltpu.VMEM((CHUNK,), jnp.float32),
            pltpu.VMEM((CHUNK,), jnp.float32),
            pltpu.VMEM((CHUNK, D), jnp.float32),
            pltpu.VMEM_SHARED((N, D), jnp.float32),
            pltpu.SemaphoreType.DMA,
            pltpu.SemaphoreType.DMA,
        ],
    )(xl, rows3, cols3, w3, dinv)


def kernel(x, edge_index, edge_embed, Wl1, W11, b11, W21, b21, bias1,
           Wl2, W12, b12, W22, b22, bias2):
    row = edge_index[0]
    col = edge_index[1]
    pad = E_PAD - E
    ipad = jnp.zeros((pad,), jnp.int32)
    rows3 = jnp.concatenate([row, ipad]).reshape(NT, CPT, CHUNK)
    colp = jnp.concatenate([col, ipad])
    cols3 = colp.reshape(NT, CPT, CHUNK)
    cols_deg = colp.reshape(NS, CPT_DEG, CHUNK)

    w1, w2 = _edge_mlp(edge_embed, W11, b11, W21, b21, W12, b12, W22, b22)
    w1 = w1[:, 0]
    w2 = w2[:, 0]
    fpad = jnp.zeros((pad,), jnp.float32)
    w1p = jnp.concatenate([w1, fpad])
    w2p = jnp.concatenate([w2, fpad])
    w_deg = jnp.stack([w1p, w2p]).reshape(NC, NS, CPT_DEG, CHUNK)

    deg = _deg(w_deg, cols_deg)
    dinv = _dinv(deg)
    dinv1 = dinv[:N]
    dinv2 = dinv[N:]

    xl1 = _node_matmul(x, Wl1)
    p1 = _seg(xl1, rows3, cols3, w1p.reshape(NT, CPT, CHUNK), dinv1)
    xl2 = _fin1(p1, xl1, dinv1.reshape(N, 1), bias1, Wl2)
    p2 = _seg(xl2, rows3, cols3, w2p.reshape(NT, CPT, CHUNK), dinv2)
    return _fin2(p2, xl2, dinv2.reshape(N, 1), bias2)


# speedup vs baseline: 10.2305x; 1.1554x over previous
import functools
import jax
import jax.numpy as jnp
from jax import lax
from jax.experimental import pallas as pl
from jax.experimental.pallas import tpu as pltpu
from jax.experimental.pallas import tpu_sc as plsc

N, E, D, H = 10000, 320000, 128, 128
NC, NS, L = 2, 16, 16
NT = NC * NS
CHUNK = 128
CPT = 80
E_PAD = NT * CPT * CHUNK
CPT_DEG = E_PAD // (NS * CHUNK)
ROWS_PER_TILE = N // NS

_mesh = plsc.VectorSubcoreMesh(core_axis_name="c", subcore_axis_name="s")


def _mlp_body(ee_ref, W11_ref, b11_ref, W21_ref, b21_ref,
              W12_ref, b12_ref, W22_ref, b22_ref, w1_ref, w2_ref):
    ee = ee_ref[...]
    dn = (((1,), (1,)), ((), ()))
    h1 = jnp.maximum(lax.dot_general(ee, W11_ref[...], dn,
                                     preferred_element_type=jnp.float32)
                     + b11_ref[...], 0.0)
    w1_ref[...] = jax.nn.sigmoid(jnp.sum(h1 * W21_ref[...], axis=1)
                                 + b21_ref[0, 0])[:, None]
    h2 = jnp.maximum(lax.dot_general(ee, W12_ref[...], dn,
                                     preferred_element_type=jnp.float32)
                     + b12_ref[...], 0.0)
    w2_ref[...] = jax.nn.sigmoid(jnp.sum(h2 * W22_ref[...], axis=1)
                                 + b22_ref[0, 0])[:, None]


def _edge_mlp(edge_embed, W11, b11, W21, b21, W12, b12, W22, b22):
    BE = 6400
    grid = E // BE
    full = pl.BlockSpec((H, H), lambda i: (0, 0))
    vec = pl.BlockSpec((1, H), lambda i: (0, 0))
    return pl.pallas_call(
        _mlp_body,
        grid=(grid,),
        in_specs=[pl.BlockSpec((BE, H), lambda i: (i, 0)),
                  full, vec, vec, pl.BlockSpec((1, 1), lambda i: (0, 0)),
                  full, vec, vec, pl.BlockSpec((1, 1), lambda i: (0, 0))],
        out_specs=[pl.BlockSpec((BE, 1), lambda i: (i, 0)),
                   pl.BlockSpec((BE, 1), lambda i: (i, 0))],
        out_shape=[jax.ShapeDtypeStruct((E, 1), jnp.float32),
                   jax.ShapeDtypeStruct((E, 1), jnp.float32)],
    )(edge_embed, W11, b11.reshape(1, H), W21, b21.reshape(1, 1),
      W12, b12.reshape(1, H), W22, b22.reshape(1, 1))


def _matmul_body(x_ref, W_ref, o_ref):
    o_ref[...] = lax.dot_general(x_ref[...], W_ref[...],
                                 (((1,), (1,)), ((), ())),
                                 preferred_element_type=jnp.float32)


def _node_matmul(x, Wl):
    return pl.pallas_call(
        _matmul_body,
        out_shape=jax.ShapeDtypeStruct((N, D), jnp.float32),
    )(x, Wl)


def _dinv_body(deg_ref, o_ref):
    o_ref[...] = lax.rsqrt(deg_ref[...] + 1.0)


def _dinv(deg):
    return pl.pallas_call(
        _dinv_body,
        out_shape=jax.ShapeDtypeStruct((NC * N,), jnp.float32),
    )(deg)


def _fin1_body(p_ref, xl_ref, dinv_ref, bias_ref, Wl2_ref, o_ref):
    dsq = dinv_ref[...] * dinv_ref[...]
    h = p_ref[0:N, :] + p_ref[N:2 * N, :] + dsq * xl_ref[...] + bias_ref[...]
    h = jnp.maximum(h, 0.0)
    o_ref[...] = lax.dot_general(h, Wl2_ref[...], (((1,), (1,)), ((), ())),
                                 preferred_element_type=jnp.float32)


def _fin1(p, xl, dinv_col, bias, Wl2):
    return pl.pallas_call(
        _fin1_body,
        out_shape=jax.ShapeDtypeStruct((N, D), jnp.float32),
    )(p, xl, dinv_col, bias.reshape(1, D), Wl2)


def _fin2_body(p_ref, xl_ref, dinv_ref, bias_ref, o_ref):
    dsq = dinv_ref[...] * dinv_ref[...]
    h = p_ref[0:N, :] + p_ref[N:2 * N, :] + dsq * xl_ref[...] + bias_ref[...]
    o_ref[...] = jnp.maximum(h + h, 0.0)


def _fin2(p, xl, dinv_col, bias):
    return pl.pallas_call(
        _fin2_body,
        out_shape=jax.ShapeDtypeStruct((N, D), jnp.float32),
    )(p, xl, dinv_col, bias.reshape(1, D))


def _deg_body(w_hbm, cols_hbm, out_hbm, cols_v, w_v, zbuf, vbuf, acc, sem):
    c = lax.axis_index("c")
    s = lax.axis_index("s")
    pltpu.sync_copy(cols_hbm.at[s], cols_v)
    pltpu.sync_copy(w_hbm.at[c, s], w_v)

    def _zb(i, carry):
        zbuf[pl.ds(i * L, L)] = jnp.zeros((L,), jnp.float32)
        return carry
    lax.fori_loop(0, 63, _zb, 0)

    @pl.when(s == 0)
    def _():
        def _zc(i, carry):
            pltpu.sync_copy(zbuf.at[pl.ds(0, 1000)],
                            acc.at[pl.ds(i * 1000, 1000)])
            return carry
        lax.fori_loop(0, 10, _zc, 0)

    plsc.subcore_barrier()

    def _sc(j, carry):
        pltpu.sync_copy(w_v.at[j], acc.at[cols_v.at[j]], add=True)
        return carry
    lax.fori_loop(0, CPT_DEG, _sc, 0)

    plsc.subcore_barrier()

    @pl.when(s == 0)
    def _():
        pltpu.sync_copy(acc, vbuf)
        pltpu.sync_copy(vbuf, out_hbm.at[pl.ds(c * N, N)])


def _deg(w_deg, cols_deg):
    return pl.kernel(
        _deg_body,
        out_type=jax.ShapeDtypeStruct((NC * N,), jnp.float32),
        mesh=_mesh,
        scratch_types=[
            pltpu.VMEM((CPT_DEG, CHUNK), jnp.int32),
            pltpu.VMEM((CPT_DEG, CHUNK), jnp.float32),
            pltpu.VMEM((1008,), jnp.float32),
            pltpu.VMEM((N,), jnp.float32),
            pltpu.VMEM_SHARED((N,), jnp.float32),
            pltpu.SemaphoreType.DMA,
        ],
    )(w_deg, cols_deg)


NBUF = 2
EBUF = 4


def _seg_body(xl_hbm, edata_hbm, w_hbm, dinv_hbm, out_hbm,
              ebuf, w_v, drbuf, dcbuf, nbuf, gbuf, acc,
              e0, e1, e2, e3, g0, g1, n0, n1, s0, s1):
    esems = (e0, e1, e2, e3)
    gsems = (g0, g1)
    nsems = (n0, n1)
    ssems = (s0, s1)
    c = lax.axis_index("c")
    s = lax.axis_index("s")
    wid = c * NS + s
    pltpu.sync_copy(w_hbm.at[wid], w_v)

    def _zb(i, carry):
        for k in range(D // L):
            gbuf[0, i, pl.ds(k * L, L)] = jnp.zeros((L,), jnp.float32)
        return carry
    lax.fori_loop(0, CHUNK, _zb, 0)
    base = s * 624

    @pl.when(s < NS - 1)
    def _():
        for m, ln in ((0, 128), (128, 128), (256, 128), (384, 128),
                      (512, 112)):
            pltpu.sync_copy(gbuf.at[0].at[pl.ds(0, ln)],
                            acc.at[pl.ds(base + m, ln)])

    @pl.when(s == NS - 1)
    def _():
        for m in range(5):
            pltpu.sync_copy(gbuf.at[0].at[pl.ds(0, 128)],
                            acc.at[pl.ds(base + m * 128, 128)])

    plsc.subcore_barrier()

    def _issue_edata(jn, be):
        pltpu.async_copy(edata_hbm.at[wid, jn], ebuf.at[be], esems[be])

    def _issue_gather(jn, be, bg):
        pltpu.async_copy(xl_hbm.at[ebuf.at[be, 0]], gbuf.at[bg], gsems[bg])
        pltpu.async_copy(dinv_hbm.at[ebuf.at[be, 0]], drbuf.at[bg],
                         nsems[bg])
        pltpu.async_copy(dinv_hbm.at[ebuf.at[be, 1]], dcbuf.at[bg],
                         nsems[bg])

    for jn in range(3):
        _issue_edata(jn, jn)
    pltpu.make_async_copy(edata_hbm.at[wid, 0], ebuf.at[0], esems[0]).wait()
    _issue_gather(0, 0, 0)

    def _outer(g, carry):
        for b4 in range(EBUF):
            j = g * EBUF + b4
            bg = b4 % NBUF
            bgn = (b4 + 1) % NBUF
            be = b4
            ben = (b4 + 1) % EBUF
            be3 = (b4 + 3) % EBUF

            @pl.when(j >= 1)
            def _():
                pltpu.make_async_copy(gbuf.at[bgn], acc.at[ebuf.at[be, 1]],
                                      ssems[bgn]).wait()

            @pl.when(j + 3 < CPT)
            def _():
                _issue_edata(j + 3, be3)

            @pl.when(j + 1 < CPT)
            def _():
                pltpu.make_async_copy(edata_hbm.at[wid, 0], ebuf.at[ben],
                                      esems[ben]).wait()
                _issue_gather(j + 1, ben, bgn)

            pltpu.make_async_copy(dinv_hbm.at[ebuf.at[be, 0]],
                                  drbuf.at[bg], nsems[bg]).wait()
            pltpu.make_async_copy(dinv_hbm.at[ebuf.at[be, 1]],
                                  dcbuf.at[bg], nsems[bg]).wait()
            for k in range(CHUNK // L):
                sl = pl.ds(k * L, L)
                nbuf[sl] = w_v[j, sl] * drbuf[bg, sl] * dcbuf[bg, sl]

            pltpu.make_async_copy(xl_hbm.at[ebuf.at[be, 0]], gbuf.at[bg],
                                  gsems[bg]).wait()

            def _eb(gg, carry2):
                nv16 = nbuf[pl.ds(gg * L, L)]
                for l in range(L):
                    e = gg * L + l
                    nv = nv16[l]
                    for k in range(D // L):
                        sl = pl.ds(k * L, L)
                        gbuf[bg, e, sl] = gbuf[bg, e, sl] * nv
                return carry2
            lax.fori_loop(0, CHUNK // L, _eb, 0)

            pltpu.async_copy(gbuf.at[bg], acc.at[ebuf.at[be, 1]],
                             ssems[bg], add=True)
        return carry
    lax.fori_loop(0, CPT // EBUF, _outer, 0)

    pltpu.make_async_copy(gbuf.at[(CPT - 1) % NBUF],
                          acc.at[ebuf.at[(CPT - 1) % EBUF, 1]],
                          ssems[(CPT - 1) % NBUF]).wait()

    plsc.subcore_barrier()

    @pl.when(s < NS - 1)
    def _():
        pltpu.sync_copy(acc.at[pl.ds(base, 624)],
                        out_hbm.at[pl.ds(c * N + base, 624)])

    @pl.when(s == NS - 1)
    def _():
        pltpu.sync_copy(acc.at[pl.ds(base, 640)],
                        out_hbm.at[pl.ds(c * N + base, 640)])


def _seg(xl, edata, w3, dinv):
    return pl.kernel(
        _seg_body,
        out_type=jax.ShapeDtypeStruct((NC * N, D), jnp.float32),
        mesh=_mesh,
        scratch_types=[
            pltpu.VMEM((EBUF, 2, CHUNK), jnp.int32),
            pltpu.VMEM((CPT, CHUNK), jnp.float32),
            pltpu.VMEM((NBUF, CHUNK), jnp.float32),
            pltpu.VMEM((NBUF, CHUNK), jnp.float32),
            pltpu.VMEM((CHUNK,), jnp.float32),
            pltpu.VMEM((NBUF, CHUNK, D), jnp.float32),
            pltpu.VMEM_SHARED((N, D), jnp.float32),
        ] + [pltpu.SemaphoreType.DMA] * (EBUF + 2 * NBUF + 2),
    )(xl, edata, w3, dinv)


def kernel(x, edge_index, edge_embed, Wl1, W11, b11, W21, b21, bias1,
           Wl2, W12, b12, W22, b22, bias2):
    row = edge_index[0]
    col = edge_index[1]
    pad = E_PAD - E
    ipad = jnp.zeros((pad,), jnp.int32)
    rowp = jnp.concatenate([row, ipad])
    colp = jnp.concatenate([col, ipad])
    cols_deg = colp.reshape(NS, CPT_DEG, CHUNK)

    w1, w2 = _edge_mlp(edge_embed, W11, b11, W21, b21, W12, b12, W22, b22)
    w1 = w1[:, 0]
    w2 = w2[:, 0]
    fpad = jnp.zeros((pad,), jnp.float32)
    w1p = jnp.concatenate([w1, fpad])
    w2p = jnp.concatenate([w2, fpad])
    w_deg = jnp.stack([w1p, w2p]).reshape(NC, NS, CPT_DEG, CHUNK)

    edata = jnp.stack(
        [rowp.reshape(NT, CPT, CHUNK), colp.reshape(NT, CPT, CHUNK)],
        axis=2)

    deg = _deg(w_deg, cols_deg)
    dinv = _dinv(deg)
    dinv1 = dinv[:N]
    dinv2 = dinv[N:]

    xl1 = _node_matmul(x, Wl1)
    p1 = _seg(xl1, edata, w1p.reshape(NT, CPT, CHUNK), dinv1)
    xl2 = _fin1(p1, xl1, dinv1.reshape(N, 1), bias1, Wl2)
    p2 = _seg(xl2, edata, w2p.reshape(NT, CPT, CHUNK), dinv2)
    return _fin2(p2, xl2, dinv2.reshape(N, 1), bias2)
